# baseline (speedup 1.0000x reference)
import functools

import jax
import jax.numpy as jnp
import numpy as np
from jax import lax
from jax.experimental import pallas as pl
from jax.experimental.pallas import tpu as pltpu
from jax.experimental.pallas import tpu_sc as plsc

_N = 10000
_E = 160000
_K = 4

_TILES = 32
_B = 32
_GC = 16
_C0 = 0
_C1 = 20
_TCH = 16 * (_C0 + _C1)
_EPAD = _TCH * _GC * _B
_NA = 10240
_NPT = _NA // 16

_COLMAP = np.empty(128, np.int32)
for _m in range(4):
    for _i in range(16):
        _COLMAP[32 * _m + 2 * _i] = 32 * _m + _i
        _COLMAP[32 * _m + 2 * _i + 1] = 32 * _m + 16 + _i


def _prep_body(ws_ref, a_ref, px_ref, py_ref, snd_ref, w_ref, g_ref):
    ws = ws_ref[0, 0]
    a = a_ref[0, 0]
    dx = px_ref[...] / ws
    dy = py_ref[...] / ws
    r2 = dx * dx + dy * dy
    win = jnp.power(jnp.maximum(1.0 - r2, 0.0), a)
    ux = (jnp.clip(dx, -1.0, 1.0) + 1.0) * (0.5 * (_K - 1))
    uy = (jnp.clip(dy, -1.0, 1.0) + 1.0) * (0.5 * (_K - 1))
    ix = jnp.clip(jnp.floor(ux).astype(jnp.int32), 0, _K - 2)
    iy = jnp.clip(jnp.floor(uy).astype(jnp.int32), 0, _K - 2)
    fx = ux - ix.astype(jnp.float32)
    fy = uy - iy.astype(jnp.float32)
    base = (snd_ref[...] * _K + iy) * (_K - 1) + ix
    g_ref[0] = base
    g_ref[1] = base + (_K - 1)
    c = 0
    for ddy in (0, 1):
        cy = fy if ddy == 1 else 1.0 - fy
        for ddx in (0, 1):
            cx = fx if ddx == 1 else 1.0 - fx
            w_ref[c] = win * cy * cx
            c += 1


def _edge_prep(rel_pos, senders, ws, a):
    er = _E // 128
    py = rel_pos[:, 0].reshape(er, 128)
    px = rel_pos[:, 1].reshape(er, 128)
    snd = senders.astype(jnp.int32).reshape(er, 128)
    ws_s = jnp.asarray(ws, jnp.float32).reshape(1, 1)
    a_s = jnp.asarray(a, jnp.float32).reshape(1, 1)
    w4, g4 = pl.pallas_call(
        _prep_body,
        in_specs=[
            pl.BlockSpec(memory_space=pltpu.SMEM),
            pl.BlockSpec(memory_space=pltpu.SMEM),
            pl.BlockSpec((er, 128), lambda: (0, 0)),
            pl.BlockSpec((er, 128), lambda: (0, 0)),
            pl.BlockSpec((er, 128), lambda: (0, 0)),
        ],
        out_specs=[
            pl.BlockSpec((4, er, 128), lambda: (0, 0, 0)),
            pl.BlockSpec((2, er, 128), lambda: (0, 0, 0)),
        ],
        out_shape=[
            jax.ShapeDtypeStruct((4, er, 128), jnp.float32),
            jax.ShapeDtypeStruct((2, er, 128), jnp.int32),
        ],
    )(ws_s, a_s, px, py, snd)
    wE = w4.transpose(1, 2, 0).reshape(_E * 4)
    gE = g4.transpose(1, 2, 0).reshape(_E * 2)
    wE = jnp.pad(wE, (0, _EPAD * 4 - _E * 4)).reshape(_TCH, _GC, 4 * _B)
    gE = jnp.pad(gE, (0, _EPAD * 2 - _E * 2)).reshape(_TCH, _GC, 2 * _B)
    return wE, gE


def _packed_weights(W):
    Wc = W[..., _COLMAP]
    wp = jnp.concatenate([Wc[:, 0:_K - 1], Wc[:, 1:_K]], axis=-1)
    cin = W.shape[2]
    w2d = wp.transpose(2, 0, 1, 3).reshape(cin, _K * (_K - 1) * 256)
    return w2d[:, 0::2].astype(jnp.bfloat16), w2d[:, 1::2].astype(jnp.bfloat16)


def _mm_body(f_ref, wa_ref, wb_ref, o_ref):
    f16 = f_ref[...].astype(jnp.bfloat16)
    a = jnp.dot(f16, wa_ref[...], preferred_element_type=jnp.float32)
    b = jnp.dot(f16, wb_ref[...], preferred_element_type=jnp.float32)
    ai = lax.bitcast_convert_type(a, jnp.int32)
    bi = lax.bitcast_convert_type(b, jnp.int32)
    o_ref[...] = (((ai + 0x8000) >> 16) & 0xFFFF) | ((bi + 0x8000) & -65536)


def _tap_transform(feat, wab):
    wa, wb = wab
    np_, cin = feat.shape
    o2 = wa.shape[1]
    bn = 512
    h = pl.pallas_call(
        _mm_body,
        grid=(np_ // bn,),
        in_specs=[pl.BlockSpec((bn, cin), lambda i: (i, 0)),
                  pl.BlockSpec((cin, o2), lambda i: (0, 0)),
                  pl.BlockSpec((cin, o2), lambda i: (0, 0))],
        out_specs=pl.BlockSpec((bn, o2), lambda i: (i, 0)),
        out_shape=jax.ShapeDtypeStruct((np_, o2), jnp.int32),
    )(feat, wa, wb)
    return h.reshape(np_ * _K * (_K - 1), 128)


def _bn_relu_body(p_ref, sc_ref, of_ref, o_ref):
    s = p_ref[0] + p_ref[1]
    m = jnp.mean(s, axis=0, keepdims=True)
    v = jnp.mean(s * s, axis=0, keepdims=True) - m * m
    z = sc_ref[...] * (s - m) * lax.rsqrt(v + 1e-5) + of_ref[...]
    o_ref[...] = jnp.maximum(z, 0.0)


def _bn_relu(p, scale, offset):
    return pl.pallas_call(
        _bn_relu_body,
        in_specs=[pl.BlockSpec((2, _N, 128), lambda: (0, 0, 0)),
                  pl.BlockSpec((1, 128), lambda: (0, 0)),
                  pl.BlockSpec((1, 128), lambda: (0, 0))],
        out_specs=pl.BlockSpec((_N, 128), lambda: (0, 0)),
        out_shape=jax.ShapeDtypeStruct((_N, 128), jnp.float32),
    )(p, scale.reshape(1, 128), offset.reshape(1, 128))


def _final_body(p_ref, sc_ref, of_ref, x_ref, y_ref, o_ref):
    s = p_ref[0] + p_ref[1]
    m = jnp.mean(s, axis=0, keepdims=True)
    v = jnp.mean(s * s, axis=0, keepdims=True) - m * m
    z = sc_ref[...] * (s - m) * lax.rsqrt(v + 1e-5) + of_ref[...]
    wei = jax.nn.sigmoid(z)
    o_ref[...] = 2.0 * x_ref[...] * wei + 2.0 * y_ref[...] * (1.0 - wei)


def _finalize(p, scale, offset, x, y):
    return pl.pallas_call(
        _final_body,
        in_specs=[pl.BlockSpec((2, _N, 128), lambda: (0, 0, 0)),
                  pl.BlockSpec((1, 128), lambda: (0, 0)),
                  pl.BlockSpec((1, 128), lambda: (0, 0)),
                  pl.BlockSpec((_N, 128), lambda: (0, 0)),
                  pl.BlockSpec((_N, 128), lambda: (0, 0))],
        out_specs=pl.BlockSpec((_N, 128), lambda: (0, 0)),
        out_shape=jax.ShapeDtypeStruct((_N, 128), jnp.float32),
    )(p, scale.reshape(1, 128), offset.reshape(1, 128), x, y)


def _sc_scatter_body(gidx_hbm, w_hbm, recv_hbm, table_hbm, out_hbm,
                     gidx_v, w_v, recv_v, g0_v, g1_v, g2_v, g3_v, c0_v, c1_v,
                     acc_sh, sem_m, sem_g0, sem_g1, sem_g2, sem_g3,
                     sem_s0, sem_s1):
    c = lax.axis_index("c")
    s = lax.axis_index("s")
    cnt = jnp.where(c == 0, _C0, _C1)
    base = jnp.where(c == 0, s * _C0, 16 * _C0 + s * _C1)

    zvec = jnp.zeros((16,), jnp.float32)
    for i in range(_B):
        for j in range(8):
            c0_v[i, pl.ds(j * 16, 16)] = zvec

    def zbody(i, carry):
        pltpu.sync_copy(c0_v, acc_sh.at[pl.ds(s * _NPT + i * _B, _B)])
        return carry
    lax.fori_loop(0, _NPT // _B, zbody, 0)
    plsc.subcore_barrier()

    gbufs = ((g0_v, sem_g0), (g1_v, sem_g1), (g2_v, sem_g2), (g3_v, sem_g3))
    cbufs = ((c0_v, sem_s0), (c1_v, sem_s1))

    def compute(g, gath_v, comb_v):
        def qbody(q, qcarry):
            wq = w_v[g, pl.ds(16 * q, 16)]
            for t in range(4):
                e = 4 * q + t
                r = 8 * q + 2 * t
                w0, w1, w2, w3 = wq[4 * t], wq[4 * t + 1], wq[4 * t + 2], wq[4 * t + 3]
                for m in range(4):
                    al = gath_v[r, pl.ds(16 * m, 16)]
                    ar = gath_v[r, pl.ds(64 + 16 * m, 16)]
                    bl = gath_v[r + 1, pl.ds(16 * m, 16)]
                    br = gath_v[r + 1, pl.ds(64 + 16 * m, 16)]
                    lo = (w0 * plsc.bitcast(al << 16, jnp.float32)
                          + w1 * plsc.bitcast(ar << 16, jnp.float32)
                          + w2 * plsc.bitcast(bl << 16, jnp.float32)
                          + w3 * plsc.bitcast(br << 16, jnp.float32))
                    hi = (w0 * plsc.bitcast(al & -65536, jnp.float32)
                          + w1 * plsc.bitcast(ar & -65536, jnp.float32)
                          + w2 * plsc.bitcast(bl & -65536, jnp.float32)
                          + w3 * plsc.bitcast(br & -65536, jnp.float32))
                    comb_v[e, pl.ds(32 * m, 16)] = lo
                    comb_v[e, pl.ds(32 * m + 16, 16)] = hi
            return qcarry
        lax.fori_loop(0, _B // 4, qbody, 0)

    def drain_gather(gath_v, sem):
        pltpu.make_async_copy(table_hbm.at[pl.ds(0, 2 * _B)], gath_v, sem).wait()

    def drain_scatter(comb_v, sem):
        pltpu.make_async_copy(out_hbm.at[0, pl.ds(0, _B)], comb_v, sem).wait()

    def chunk_body(ch, carry):
        cidx = base + ch
        pltpu.async_copy(gidx_hbm.at[cidx], gidx_v, sem_m)
        pltpu.async_copy(w_hbm.at[cidx], w_v, sem_m)
        pltpu.async_copy(recv_hbm.at[cidx], recv_v, sem_m)
        pltpu.make_async_copy(gidx_hbm.at[cidx], gidx_v, sem_m).wait()
        pltpu.make_async_copy(w_hbm.at[cidx], w_v, sem_m).wait()
        pltpu.make_async_copy(recv_hbm.at[cidx], recv_v, sem_m).wait()

        for i, (gv, sg) in enumerate(gbufs):
            pltpu.async_copy(table_hbm.at[gidx_v.at[i]], gv, sg)

        def quad_body(q, qcarry):
            for i, (gv, sg) in enumerate(gbufs):
                g = 4 * q + i
                cv, ss = cbufs[i % 2]
                drain_gather(gv, sg)
                if i < 2:
                    @pl.when(q > 0)
                    def _():
                        drain_scatter(cv, ss)
                else:
                    drain_scatter(cv, ss)
                compute(g, gv, cv)

                @pl.when(g + 4 < _GC)
                def _():
                    pltpu.async_copy(table_hbm.at[gidx_v.at[g + 4]], gv, sg)
                pltpu.async_copy(cv, acc_sh.at[recv_v.at[g]], ss, add=True)
            return qcarry
        lax.fori_loop(0, _GC // 4, quad_body, 0)
        drain_scatter(c0_v, sem_s0)
        drain_scatter(c1_v, sem_s1)
        return carry
    lax.fori_loop(0, cnt, chunk_body, 0)

    plsc.subcore_barrier()
    pltpu.sync_copy(acc_sh.at[pl.ds(s * _NPT, _NPT)],
                    out_hbm.at[c, pl.ds(s * _NPT, _NPT)])


def _sc_scatter(gidx, w, recv, table):
    mesh = plsc.VectorSubcoreMesh(core_axis_name="c", subcore_axis_name="s")
    fn = functools.partial(
        pl.kernel,
        mesh=mesh,
        compiler_params=pltpu.CompilerParams(needs_layout_passes=False),
        out_type=jax.ShapeDtypeStruct((2, _NA, 128), jnp.float32),
        scratch_types=[
            pltpu.VMEM((_GC, 2 * _B), jnp.int32),
            pltpu.VMEM((_GC, 4 * _B), jnp.float32),
            pltpu.VMEM((_GC, _B), jnp.int32),
            pltpu.VMEM((2 * _B, 128), jnp.int32),
            pltpu.VMEM((2 * _B, 128), jnp.int32),
            pltpu.VMEM((2 * _B, 128), jnp.int32),
            pltpu.VMEM((2 * _B, 128), jnp.int32),
            pltpu.VMEM((_B, 128), jnp.float32),
            pltpu.VMEM((_B, 128), jnp.float32),
            pltpu.VMEM_SHARED((_NA, 128), jnp.float32),
            pltpu.SemaphoreType.DMA,
            pltpu.SemaphoreType.DMA,
            pltpu.SemaphoreType.DMA,
            pltpu.SemaphoreType.DMA,
            pltpu.SemaphoreType.DMA,
            pltpu.SemaphoreType.DMA,
            pltpu.SemaphoreType.DMA,
        ],
    )(_sc_scatter_body)
    return fn(gidx, w, recv, table)


def kernel(x, y, senders, receivers, rel_pos, window_support, a,
           W1, b1, bn1_scale, bn1_offset, W2, b2, bn2_scale, bn2_offset):
    n = x.shape[0]
    kk = _K * _K

    wE, gE = _edge_prep(rel_pos, senders, window_support, a)
    recv = jnp.pad(receivers.astype(jnp.int32),
                   (0, _EPAD - _E)).reshape(_TCH, _GC, _B)

    xa = jnp.concatenate([x, y], axis=-1)
    np1 = 10240
    xa_p = jnp.pad(xa, ((0, np1 - n), (0, 0)))
    w1_2d = _packed_weights(W1)
    h1 = _tap_transform(xa_p, w1_2d)
    p1 = _sc_scatter(gE, wE, recv, h1)[:, :n, :]
    xl = _bn_relu(p1, bn1_scale, bn1_offset)

    xl_p = jnp.pad(xl, ((0, np1 - n), (0, 0)))
    w2_2d = _packed_weights(W2)
    h2 = _tap_transform(xl_p, w2_2d)
    p2 = _sc_scatter(gE, wE, recv, h2)[:, :n, :]
    return _finalize(p2, bn2_scale, bn2_offset, x, y)


# speedup vs baseline: 1.6239x; 1.6239x over previous
import functools

import jax
import jax.numpy as jnp
import numpy as np
from jax import lax
from jax.experimental import pallas as pl
from jax.experimental.pallas import tpu as pltpu
from jax.experimental.pallas import tpu_sc as plsc

_N = 10000
_E = 160000
_K = 4

_TILES = 32
_B = 32
_GC = 16
_C0 = 14
_C1 = 6
_TCH = 16 * (_C0 + _C1)
_EPAD = _TCH * _GC * _B
_NA = 10240
_NPT = _NA // 16

_COLMAP = np.empty(128, np.int32)
for _m in range(4):
    for _i in range(16):
        _COLMAP[32 * _m + 2 * _i] = 32 * _m + _i
        _COLMAP[32 * _m + 2 * _i + 1] = 32 * _m + 16 + _i


def _prep_body(ws_ref, a_ref, px_ref, py_ref, snd_ref, w_ref, g_ref):
    ws = ws_ref[0, 0]
    a = a_ref[0, 0]
    dx = px_ref[...] / ws
    dy = py_ref[...] / ws
    r2 = dx * dx + dy * dy
    win = jnp.power(jnp.maximum(1.0 - r2, 0.0), a)
    ux = (jnp.clip(dx, -1.0, 1.0) + 1.0) * (0.5 * (_K - 1))
    uy = (jnp.clip(dy, -1.0, 1.0) + 1.0) * (0.5 * (_K - 1))
    ix = jnp.clip(jnp.floor(ux).astype(jnp.int32), 0, _K - 2)
    iy = jnp.clip(jnp.floor(uy).astype(jnp.int32), 0, _K - 2)
    fx = ux - ix.astype(jnp.float32)
    fy = uy - iy.astype(jnp.float32)
    base = (snd_ref[...] * _K + iy) * (_K - 1) + ix
    g_ref[0] = base
    g_ref[1] = base + (_K - 1)
    c = 0
    for ddy in (0, 1):
        cy = fy if ddy == 1 else 1.0 - fy
        for ddx in (0, 1):
            cx = fx if ddx == 1 else 1.0 - fx
            w_ref[c] = win * cy * cx
            c += 1


def _edge_prep(rel_pos, senders, ws, a):
    er = _E // 128
    py = rel_pos[:, 0].reshape(er, 128)
    px = rel_pos[:, 1].reshape(er, 128)
    snd = senders.astype(jnp.int32).reshape(er, 128)
    ws_s = jnp.asarray(ws, jnp.float32).reshape(1, 1)
    a_s = jnp.asarray(a, jnp.float32).reshape(1, 1)
    w4, g4 = pl.pallas_call(
        _prep_body,
        in_specs=[
            pl.BlockSpec(memory_space=pltpu.SMEM),
            pl.BlockSpec(memory_space=pltpu.SMEM),
            pl.BlockSpec((er, 128), lambda: (0, 0)),
            pl.BlockSpec((er, 128), lambda: (0, 0)),
            pl.BlockSpec((er, 128), lambda: (0, 0)),
        ],
        out_specs=[
            pl.BlockSpec((4, er, 128), lambda: (0, 0, 0)),
            pl.BlockSpec((2, er, 128), lambda: (0, 0, 0)),
        ],
        out_shape=[
            jax.ShapeDtypeStruct((4, er, 128), jnp.float32),
            jax.ShapeDtypeStruct((2, er, 128), jnp.int32),
        ],
    )(ws_s, a_s, px, py, snd)
    wE = w4.transpose(1, 2, 0).reshape(_E * 4)
    gE = g4.transpose(1, 2, 0).reshape(_E * 2)
    wE = jnp.pad(wE, (0, _EPAD * 4 - _E * 4)).reshape(_TCH, _GC, 4 * _B)
    gE = jnp.pad(gE, (0, _EPAD * 2 - _E * 2)).reshape(_TCH, _GC, 2 * _B)
    return wE, gE


def _packed_weights(W):
    Wc = W[..., _COLMAP]
    wp = jnp.concatenate([Wc[:, 0:_K - 1], Wc[:, 1:_K]], axis=-1)
    cin = W.shape[2]
    w2d = wp.transpose(2, 0, 1, 3).reshape(cin, _K * (_K - 1) * 256)
    return w2d[:, 0::2].astype(jnp.bfloat16), w2d[:, 1::2].astype(jnp.bfloat16)


def _mm_body(f_ref, wa_ref, wb_ref, o_ref):
    f16 = f_ref[...].astype(jnp.bfloat16)
    a = jnp.dot(f16, wa_ref[...], preferred_element_type=jnp.float32)
    b = jnp.dot(f16, wb_ref[...], preferred_element_type=jnp.float32)
    ai = lax.bitcast_convert_type(a, jnp.int32)
    bi = lax.bitcast_convert_type(b, jnp.int32)
    w = (((ai + 0x8000) >> 16) & 0xFFFF) | ((bi + 0x8000) & -65536)
    o_ref[...] = w.reshape(o_ref.shape)


def _tap_transform(feat, wab):
    wa, wb = wab
    n, cin = feat.shape
    np_ = 10240
    kk2 = _K * (_K - 1)
    o2 = wa.shape[1]
    bn = 512
    return pl.pallas_call(
        _mm_body,
        grid=(np_ // bn,),
        in_specs=[pl.BlockSpec((bn, cin), lambda i: (i, 0)),
                  pl.BlockSpec((cin, o2), lambda i: (0, 0)),
                  pl.BlockSpec((cin, o2), lambda i: (0, 0))],
        out_specs=pl.BlockSpec((bn * kk2, 128), lambda i: (i, 0)),
        out_shape=jax.ShapeDtypeStruct((np_ * kk2, 128), jnp.int32),
    )(feat, wa, wb)


def _bn_relu_body(p_ref, sc_ref, of_ref, o_ref):
    s = p_ref[0] + p_ref[1]
    m = jnp.mean(s, axis=0, keepdims=True)
    v = jnp.mean(s * s, axis=0, keepdims=True) - m * m
    z = sc_ref[...] * (s - m) * lax.rsqrt(v + 1e-5) + of_ref[...]
    o_ref[...] = jnp.maximum(z, 0.0)


def _bn_relu(p, scale, offset):
    return pl.pallas_call(
        _bn_relu_body,
        in_specs=[pl.BlockSpec((2, _N, 128), lambda: (0, 0, 0)),
                  pl.BlockSpec((1, 128), lambda: (0, 0)),
                  pl.BlockSpec((1, 128), lambda: (0, 0))],
        out_specs=pl.BlockSpec((_N, 128), lambda: (0, 0)),
        out_shape=jax.ShapeDtypeStruct((_N, 128), jnp.float32),
    )(p, scale.reshape(1, 128), offset.reshape(1, 128))


def _final_body(p_ref, sc_ref, of_ref, x_ref, y_ref, o_ref):
    s = p_ref[0] + p_ref[1]
    m = jnp.mean(s, axis=0, keepdims=True)
    v = jnp.mean(s * s, axis=0, keepdims=True) - m * m
    z = sc_ref[...] * (s - m) * lax.rsqrt(v + 1e-5) + of_ref[...]
    wei = jax.nn.sigmoid(z)
    o_ref[...] = 2.0 * x_ref[...] * wei + 2.0 * y_ref[...] * (1.0 - wei)


def _finalize(p, scale, offset, x, y):
    return pl.pallas_call(
        _final_body,
        in_specs=[pl.BlockSpec((2, _N, 128), lambda: (0, 0, 0)),
                  pl.BlockSpec((1, 128), lambda: (0, 0)),
                  pl.BlockSpec((1, 128), lambda: (0, 0)),
                  pl.BlockSpec((_N, 128), lambda: (0, 0)),
                  pl.BlockSpec((_N, 128), lambda: (0, 0))],
        out_specs=pl.BlockSpec((_N, 128), lambda: (0, 0)),
        out_shape=jax.ShapeDtypeStruct((_N, 128), jnp.float32),
    )(p, scale.reshape(1, 128), offset.reshape(1, 128), x, y)


def _sc_scatter_body(gidx_hbm, w_hbm, recv_hbm, table_hbm, out_hbm,
                     gidx_v, w_v, recv_v, g0_v, g1_v, g2_v, g3_v, c0_v, c1_v,
                     acc_sh, sem_m, sem_g0, sem_g1, sem_g2, sem_g3,
                     sem_s0, sem_s1):
    c = lax.axis_index("c")
    s = lax.axis_index("s")
    cnt = jnp.where(c == 0, _C0, _C1)
    base = jnp.where(c == 0, s * _C0, 16 * _C0 + s * _C1)

    zvec = jnp.zeros((16,), jnp.float32)
    for i in range(_B):
        for j in range(8):
            c0_v[i, pl.ds(j * 16, 16)] = zvec

    def zbody(i, carry):
        pltpu.sync_copy(c0_v, acc_sh.at[pl.ds(s * _NPT + i * _B, _B)])
        return carry
    lax.fori_loop(0, _NPT // _B, zbody, 0)
    plsc.subcore_barrier()

    gbufs = ((g0_v, sem_g0), (g1_v, sem_g1), (g2_v, sem_g2), (g3_v, sem_g3))
    cbufs = ((c0_v, sem_s0), (c1_v, sem_s1))

    def compute(g, gath_v, comb_v):
        def qbody(q, qcarry):
            wq = w_v[g, pl.ds(16 * q, 16)]
            for t in range(4):
                e = 4 * q + t
                r = 8 * q + 2 * t
                w0, w1, w2, w3 = wq[4 * t], wq[4 * t + 1], wq[4 * t + 2], wq[4 * t + 3]
                for m in range(4):
                    al = gath_v[r, pl.ds(16 * m, 16)]
                    ar = gath_v[r, pl.ds(64 + 16 * m, 16)]
                    bl = gath_v[r + 1, pl.ds(16 * m, 16)]
                    br = gath_v[r + 1, pl.ds(64 + 16 * m, 16)]
                    lo = (w0 * plsc.bitcast(al << 16, jnp.float32)
                          + w1 * plsc.bitcast(ar << 16, jnp.float32)
                          + w2 * plsc.bitcast(bl << 16, jnp.float32)
                          + w3 * plsc.bitcast(br << 16, jnp.float32))
                    hi = (w0 * plsc.bitcast(al & -65536, jnp.float32)
                          + w1 * plsc.bitcast(ar & -65536, jnp.float32)
                          + w2 * plsc.bitcast(bl & -65536, jnp.float32)
                          + w3 * plsc.bitcast(br & -65536, jnp.float32))
                    comb_v[e, pl.ds(32 * m, 16)] = lo
                    comb_v[e, pl.ds(32 * m + 16, 16)] = hi
            return qcarry
        lax.fori_loop(0, _B // 4, qbody, 0)

    def drain_gather(gath_v, sem):
        pltpu.make_async_copy(table_hbm.at[pl.ds(0, 2 * _B)], gath_v, sem).wait()

    def drain_scatter(comb_v, sem):
        pltpu.make_async_copy(out_hbm.at[0, pl.ds(0, _B)], comb_v, sem).wait()

    def chunk_body(ch, carry):
        cidx = base + ch
        pltpu.async_copy(gidx_hbm.at[cidx], gidx_v, sem_m)
        pltpu.async_copy(w_hbm.at[cidx], w_v, sem_m)
        pltpu.async_copy(recv_hbm.at[cidx], recv_v, sem_m)
        pltpu.make_async_copy(gidx_hbm.at[cidx], gidx_v, sem_m).wait()
        pltpu.make_async_copy(w_hbm.at[cidx], w_v, sem_m).wait()
        pltpu.make_async_copy(recv_hbm.at[cidx], recv_v, sem_m).wait()

        for i, (gv, sg) in enumerate(gbufs):
            pltpu.async_copy(table_hbm.at[gidx_v.at[i]], gv, sg)

        def quad_body(q, qcarry):
            for i, (gv, sg) in enumerate(gbufs):
                g = 4 * q + i
                cv, ss = cbufs[i % 2]
                drain_gather(gv, sg)
                if i < 2:
                    @pl.when(q > 0)
                    def _():
                        drain_scatter(cv, ss)
                else:
                    drain_scatter(cv, ss)
                compute(g, gv, cv)

                @pl.when(g + 4 < _GC)
                def _():
                    pltpu.async_copy(table_hbm.at[gidx_v.at[g + 4]], gv, sg)
                pltpu.async_copy(cv, acc_sh.at[recv_v.at[g]], ss, add=True)
            return qcarry
        lax.fori_loop(0, _GC // 4, quad_body, 0)
        drain_scatter(c0_v, sem_s0)
        drain_scatter(c1_v, sem_s1)
        return carry
    lax.fori_loop(0, cnt, chunk_body, 0)

    plsc.subcore_barrier()
    pltpu.sync_copy(acc_sh.at[pl.ds(s * _NPT, _NPT)],
                    out_hbm.at[c, pl.ds(s * _NPT, _NPT)])


def _sc_scatter(gidx, w, recv, table):
    mesh = plsc.VectorSubcoreMesh(core_axis_name="c", subcore_axis_name="s")
    fn = functools.partial(
        pl.kernel,
        mesh=mesh,
        compiler_params=pltpu.CompilerParams(needs_layout_passes=False),
        out_type=jax.ShapeDtypeStruct((2, _NA, 128), jnp.float32),
        scratch_types=[
            pltpu.VMEM((_GC, 2 * _B), jnp.int32),
            pltpu.VMEM((_GC, 4 * _B), jnp.float32),
            pltpu.VMEM((_GC, _B), jnp.int32),
            pltpu.VMEM((2 * _B, 128), jnp.int32),
            pltpu.VMEM((2 * _B, 128), jnp.int32),
            pltpu.VMEM((2 * _B, 128), jnp.int32),
            pltpu.VMEM((2 * _B, 128), jnp.int32),
            pltpu.VMEM((_B, 128), jnp.float32),
            pltpu.VMEM((_B, 128), jnp.float32),
            pltpu.VMEM_SHARED((_NA, 128), jnp.float32),
            pltpu.SemaphoreType.DMA,
            pltpu.SemaphoreType.DMA,
            pltpu.SemaphoreType.DMA,
            pltpu.SemaphoreType.DMA,
            pltpu.SemaphoreType.DMA,
            pltpu.SemaphoreType.DMA,
            pltpu.SemaphoreType.DMA,
        ],
    )(_sc_scatter_body)
    return fn(gidx, w, recv, table)


def kernel(x, y, senders, receivers, rel_pos, window_support, a,
           W1, b1, bn1_scale, bn1_offset, W2, b2, bn2_scale, bn2_offset):
    n = x.shape[0]
    kk = _K * _K

    wE, gE = _edge_prep(rel_pos, senders, window_support, a)
    recv = jnp.pad(receivers.astype(jnp.int32),
                   (0, _EPAD - _E)).reshape(_TCH, _GC, _B)

    xa = jnp.concatenate([x, y], axis=-1)
    w1_2d = _packed_weights(W1)
    h1 = _tap_transform(xa, w1_2d)
    p1 = _sc_scatter(gE, wE, recv, h1)[:, :n, :]
    xl = _bn_relu(p1, bn1_scale, bn1_offset)

    w2_2d = _packed_weights(W2)
    h2 = _tap_transform(xl, w2_2d)
    p2 = _sc_scatter(gE, wE, recv, h2)[:, :n, :]
    return _finalize(p2, bn2_scale, bn2_offset, x, y)


# speedup vs baseline: 1.7834x; 1.0983x over previous
import functools

import jax
import jax.numpy as jnp
import numpy as np
from jax import lax
from jax.experimental import pallas as pl
from jax.experimental.pallas import tpu as pltpu
from jax.experimental.pallas import tpu_sc as plsc

_N = 10000
_E = 160000
_K = 4

_TILES = 32
_B = 32
_GC = 16
_C0 = 14
_C1 = 6
_TCH = 16 * (_C0 + _C1)
_EPAD = _TCH * _GC * _B
_NA = 10240
_NPT = _NA // 16

_COLMAP = np.empty(128, np.int32)
for _m in range(4):
    for _i in range(16):
        _COLMAP[32 * _m + 2 * _i] = 32 * _m + _i
        _COLMAP[32 * _m + 2 * _i + 1] = 32 * _m + 16 + _i


def _prep_body(ws_ref, a_ref, px_ref, py_ref, snd_ref, w_ref, g_ref):
    ws = ws_ref[0, 0]
    a = a_ref[0, 0]
    dx = px_ref[...] / ws
    dy = py_ref[...] / ws
    r2 = dx * dx + dy * dy
    win = jnp.power(jnp.maximum(1.0 - r2, 0.0), a)
    ux = (jnp.clip(dx, -1.0, 1.0) + 1.0) * (0.5 * (_K - 1))
    uy = (jnp.clip(dy, -1.0, 1.0) + 1.0) * (0.5 * (_K - 1))
    ix = jnp.clip(jnp.floor(ux).astype(jnp.int32), 0, _K - 2)
    iy = jnp.clip(jnp.floor(uy).astype(jnp.int32), 0, _K - 2)
    fx = ux - ix.astype(jnp.float32)
    fy = uy - iy.astype(jnp.float32)
    base = (snd_ref[...] * _K + iy) * (_K - 1) + ix
    g_ref[0] = base
    g_ref[1] = base + (_K - 1)
    c = 0
    for ddy in (0, 1):
        cy = fy if ddy == 1 else 1.0 - fy
        for ddx in (0, 1):
            cx = fx if ddx == 1 else 1.0 - fx
            w_ref[c] = win * cy * cx
            c += 1


def _edge_prep(rel_pos, senders, ws, a):
    er = _E // 128
    py = rel_pos[:, 0].reshape(er, 128)
    px = rel_pos[:, 1].reshape(er, 128)
    snd = senders.astype(jnp.int32).reshape(er, 128)
    ws_s = jnp.asarray(ws, jnp.float32).reshape(1, 1)
    a_s = jnp.asarray(a, jnp.float32).reshape(1, 1)
    w4, g4 = pl.pallas_call(
        _prep_body,
        in_specs=[
            pl.BlockSpec(memory_space=pltpu.SMEM),
            pl.BlockSpec(memory_space=pltpu.SMEM),
            pl.BlockSpec((er, 128), lambda: (0, 0)),
            pl.BlockSpec((er, 128), lambda: (0, 0)),
            pl.BlockSpec((er, 128), lambda: (0, 0)),
        ],
        out_specs=[
            pl.BlockSpec((4, er, 128), lambda: (0, 0, 0)),
            pl.BlockSpec((2, er, 128), lambda: (0, 0, 0)),
        ],
        out_shape=[
            jax.ShapeDtypeStruct((4, er, 128), jnp.float32),
            jax.ShapeDtypeStruct((2, er, 128), jnp.int32),
        ],
    )(ws_s, a_s, px, py, snd)
    wE = w4.transpose(1, 2, 0).reshape(_E * 4)
    gE = g4.transpose(1, 2, 0).reshape(_E * 2)
    wE = jnp.pad(wE, (0, _EPAD * 4 - _E * 4)).reshape(_TCH, _GC, 4 * _B)
    gE = jnp.pad(gE, (0, _EPAD * 2 - _E * 2)).reshape(_TCH, _GC, 2 * _B)
    return wE, gE


def _packed_weights(W):
    Wc = W[..., _COLMAP]
    wp = jnp.concatenate([Wc[:, 0:_K - 1], Wc[:, 1:_K]], axis=-1)
    cin = W.shape[2]
    w2d = wp.transpose(2, 0, 1, 3).reshape(cin, _K * (_K - 1) * 256)
    return w2d[:, 0::2].astype(jnp.bfloat16), w2d[:, 1::2].astype(jnp.bfloat16)


def _mm_body(f_ref, wa_ref, wb_ref, o_ref):
    f16 = f_ref[...].astype(jnp.bfloat16)
    a = jnp.dot(f16, wa_ref[...], preferred_element_type=jnp.float32)
    b = jnp.dot(f16, wb_ref[...], preferred_element_type=jnp.float32)
    _mm_pack(a, b, o_ref)


def _mm_pack(a, b, o_ref):
    ai = lax.bitcast_convert_type(a, jnp.int32)
    bi = lax.bitcast_convert_type(b, jnp.int32)
    w = (((ai + 0x8000) >> 16) & 0xFFFF) | ((bi + 0x8000) & -65536)
    o_ref[...] = w.reshape(o_ref.shape)


def _mm2_body(x_ref, y_ref, wax_ref, way_ref, wbx_ref, wby_ref, o_ref):
    x16 = x_ref[...].astype(jnp.bfloat16)
    y16 = y_ref[...].astype(jnp.bfloat16)
    a = (jnp.dot(x16, wax_ref[...], preferred_element_type=jnp.float32)
         + jnp.dot(y16, way_ref[...], preferred_element_type=jnp.float32))
    b = (jnp.dot(x16, wbx_ref[...], preferred_element_type=jnp.float32)
         + jnp.dot(y16, wby_ref[...], preferred_element_type=jnp.float32))
    _mm_pack(a, b, o_ref)


def _tap_transform2(x, y, wab):
    wa, wb = wab
    n, cin = x.shape
    np_ = 10240
    kk2 = _K * (_K - 1)
    o2 = wa.shape[1]
    bn = 512
    return pl.pallas_call(
        _mm2_body,
        grid=(np_ // bn,),
        in_specs=[pl.BlockSpec((bn, cin), lambda i: (i, 0)),
                  pl.BlockSpec((bn, cin), lambda i: (i, 0)),
                  pl.BlockSpec((cin, o2), lambda i: (0, 0)),
                  pl.BlockSpec((cin, o2), lambda i: (0, 0)),
                  pl.BlockSpec((cin, o2), lambda i: (0, 0)),
                  pl.BlockSpec((cin, o2), lambda i: (0, 0))],
        out_specs=pl.BlockSpec((bn * kk2, 128), lambda i: (i, 0)),
        out_shape=jax.ShapeDtypeStruct((np_ * kk2, 128), jnp.int32),
    )(x, y, wa[:cin], wa[cin:], wb[:cin], wb[cin:])


def _tap_transform(feat, wab):
    wa, wb = wab
    n, cin = feat.shape
    np_ = 10240
    kk2 = _K * (_K - 1)
    o2 = wa.shape[1]
    bn = 512
    return pl.pallas_call(
        _mm_body,
        grid=(np_ // bn,),
        in_specs=[pl.BlockSpec((bn, cin), lambda i: (i, 0)),
                  pl.BlockSpec((cin, o2), lambda i: (0, 0)),
                  pl.BlockSpec((cin, o2), lambda i: (0, 0))],
        out_specs=pl.BlockSpec((bn * kk2, 128), lambda i: (i, 0)),
        out_shape=jax.ShapeDtypeStruct((np_ * kk2, 128), jnp.int32),
    )(feat, wa, wb)


def _bn_relu_body(p_ref, sc_ref, of_ref, o_ref):
    s = p_ref[0] + p_ref[1]
    m = jnp.mean(s, axis=0, keepdims=True)
    v = jnp.mean(s * s, axis=0, keepdims=True) - m * m
    z = sc_ref[...] * (s - m) * lax.rsqrt(v + 1e-5) + of_ref[...]
    o_ref[...] = jnp.maximum(z, 0.0)


def _bn_relu(p, scale, offset):
    return pl.pallas_call(
        _bn_relu_body,
        in_specs=[pl.BlockSpec((2, _N, 128), lambda: (0, 0, 0)),
                  pl.BlockSpec((1, 128), lambda: (0, 0)),
                  pl.BlockSpec((1, 128), lambda: (0, 0))],
        out_specs=pl.BlockSpec((_N, 128), lambda: (0, 0)),
        out_shape=jax.ShapeDtypeStruct((_N, 128), jnp.float32),
    )(p, scale.reshape(1, 128), offset.reshape(1, 128))


def _final_body(p_ref, sc_ref, of_ref, x_ref, y_ref, o_ref):
    s = p_ref[0] + p_ref[1]
    m = jnp.mean(s, axis=0, keepdims=True)
    v = jnp.mean(s * s, axis=0, keepdims=True) - m * m
    z = sc_ref[...] * (s - m) * lax.rsqrt(v + 1e-5) + of_ref[...]
    wei = jax.nn.sigmoid(z)
    o_ref[...] = 2.0 * x_ref[...] * wei + 2.0 * y_ref[...] * (1.0 - wei)


def _finalize(p, scale, offset, x, y):
    return pl.pallas_call(
        _final_body,
        in_specs=[pl.BlockSpec((2, _N, 128), lambda: (0, 0, 0)),
                  pl.BlockSpec((1, 128), lambda: (0, 0)),
                  pl.BlockSpec((1, 128), lambda: (0, 0)),
                  pl.BlockSpec((_N, 128), lambda: (0, 0)),
                  pl.BlockSpec((_N, 128), lambda: (0, 0))],
        out_specs=pl.BlockSpec((_N, 128), lambda: (0, 0)),
        out_shape=jax.ShapeDtypeStruct((_N, 128), jnp.float32),
    )(p, scale.reshape(1, 128), offset.reshape(1, 128), x, y)


def _sc_scatter_body(c0, c1, gidx_hbm, w_hbm, recv_hbm, table_hbm, out_hbm,
                     gidx_v, w_v, recv_v, g0_v, g1_v, g2_v, g3_v, c0_v, c1_v,
                     acc_sh, sem_m, sem_g0, sem_g1, sem_g2, sem_g3,
                     sem_s0, sem_s1):
    c = lax.axis_index("c")
    s = lax.axis_index("s")
    cnt = jnp.where(c == 0, c0, c1)
    base = jnp.where(c == 0, s * c0, 16 * c0 + s * c1)

    zvec = jnp.zeros((16,), jnp.float32)
    for i in range(_B):
        for j in range(8):
            c0_v[i, pl.ds(j * 16, 16)] = zvec

    def zbody(i, carry):
        pltpu.sync_copy(c0_v, acc_sh.at[pl.ds(s * _NPT + i * _B, _B)])
        return carry
    lax.fori_loop(0, _NPT // _B, zbody, 0)
    plsc.subcore_barrier()

    gbufs = ((g0_v, sem_g0), (g1_v, sem_g1), (g2_v, sem_g2), (g3_v, sem_g3))
    cbufs = ((c0_v, sem_s0), (c1_v, sem_s1))

    def compute(g, gath_v, comb_v):
        def qbody(q, qcarry):
            wq = w_v[g, pl.ds(16 * q, 16)]
            for t in range(4):
                e = 4 * q + t
                r = 8 * q + 2 * t
                w0, w1, w2, w3 = wq[4 * t], wq[4 * t + 1], wq[4 * t + 2], wq[4 * t + 3]
                for m in range(4):
                    al = gath_v[r, pl.ds(16 * m, 16)]
                    ar = gath_v[r, pl.ds(64 + 16 * m, 16)]
                    bl = gath_v[r + 1, pl.ds(16 * m, 16)]
                    br = gath_v[r + 1, pl.ds(64 + 16 * m, 16)]
                    lo = (w0 * plsc.bitcast(al << 16, jnp.float32)
                          + w1 * plsc.bitcast(ar << 16, jnp.float32)
                          + w2 * plsc.bitcast(bl << 16, jnp.float32)
                          + w3 * plsc.bitcast(br << 16, jnp.float32))
                    hi = (w0 * plsc.bitcast(al & -65536, jnp.float32)
                          + w1 * plsc.bitcast(ar & -65536, jnp.float32)
                          + w2 * plsc.bitcast(bl & -65536, jnp.float32)
                          + w3 * plsc.bitcast(br & -65536, jnp.float32))
                    comb_v[e, pl.ds(32 * m, 16)] = lo
                    comb_v[e, pl.ds(32 * m + 16, 16)] = hi
            return qcarry
        lax.fori_loop(0, _B // 4, qbody, 0)

    def drain_gather(gath_v, sem):
        pltpu.make_async_copy(table_hbm.at[pl.ds(0, 2 * _B)], gath_v, sem).wait()

    def drain_scatter(comb_v, sem):
        pltpu.make_async_copy(out_hbm.at[0, pl.ds(0, _B)], comb_v, sem).wait()

    def chunk_body(ch, carry):
        cidx = base + ch
        pltpu.async_copy(gidx_hbm.at[cidx], gidx_v, sem_m)
        pltpu.async_copy(w_hbm.at[cidx], w_v, sem_m)
        pltpu.async_copy(recv_hbm.at[cidx], recv_v, sem_m)
        pltpu.make_async_copy(gidx_hbm.at[cidx], gidx_v, sem_m).wait()
        pltpu.make_async_copy(w_hbm.at[cidx], w_v, sem_m).wait()
        pltpu.make_async_copy(recv_hbm.at[cidx], recv_v, sem_m).wait()

        for i, (gv, sg) in enumerate(gbufs):
            pltpu.async_copy(table_hbm.at[gidx_v.at[i]], gv, sg)

        def quad_body(q, qcarry):
            for i, (gv, sg) in enumerate(gbufs):
                g = 4 * q + i
                cv, ss = cbufs[i % 2]
                drain_gather(gv, sg)
                if i < 2:
                    @pl.when(q > 0)
                    def _():
                        drain_scatter(cv, ss)
                else:
                    drain_scatter(cv, ss)
                compute(g, gv, cv)

                @pl.when(g + 4 < _GC)
                def _():
                    pltpu.async_copy(table_hbm.at[gidx_v.at[g + 4]], gv, sg)
                pltpu.async_copy(cv, acc_sh.at[recv_v.at[g]], ss, add=True)
            return qcarry
        lax.fori_loop(0, _GC // 4, quad_body, 0)
        drain_scatter(c0_v, sem_s0)
        drain_scatter(c1_v, sem_s1)
        return carry
    lax.fori_loop(0, cnt, chunk_body, 0)

    plsc.subcore_barrier()
    pltpu.sync_copy(acc_sh.at[pl.ds(s * _NPT, _NPT)],
                    out_hbm.at[c, pl.ds(s * _NPT, _NPT)])


def _sc_scatter(gidx, w, recv, table, c0, c1):
    mesh = plsc.VectorSubcoreMesh(core_axis_name="c", subcore_axis_name="s")
    fn = functools.partial(
        pl.kernel,
        mesh=mesh,
        compiler_params=pltpu.CompilerParams(needs_layout_passes=False),
        out_type=jax.ShapeDtypeStruct((2, _NA, 128), jnp.float32),
        scratch_types=[
            pltpu.VMEM((_GC, 2 * _B), jnp.int32),
            pltpu.VMEM((_GC, 4 * _B), jnp.float32),
            pltpu.VMEM((_GC, _B), jnp.int32),
            pltpu.VMEM((2 * _B, 128), jnp.int32),
            pltpu.VMEM((2 * _B, 128), jnp.int32),
            pltpu.VMEM((2 * _B, 128), jnp.int32),
            pltpu.VMEM((2 * _B, 128), jnp.int32),
            pltpu.VMEM((_B, 128), jnp.float32),
            pltpu.VMEM((_B, 128), jnp.float32),
            pltpu.VMEM_SHARED((_NA, 128), jnp.float32),
            pltpu.SemaphoreType.DMA,
            pltpu.SemaphoreType.DMA,
            pltpu.SemaphoreType.DMA,
            pltpu.SemaphoreType.DMA,
            pltpu.SemaphoreType.DMA,
            pltpu.SemaphoreType.DMA,
            pltpu.SemaphoreType.DMA,
        ],
    )(functools.partial(_sc_scatter_body, c0, c1))
    return fn(gidx, w, recv, table)


def kernel(x, y, senders, receivers, rel_pos, window_support, a,
           W1, b1, bn1_scale, bn1_offset, W2, b2, bn2_scale, bn2_offset):
    n = x.shape[0]
    kk = _K * _K

    wE, gE = _edge_prep(rel_pos, senders, window_support, a)
    recv = jnp.pad(receivers.astype(jnp.int32),
                   (0, _EPAD - _E)).reshape(_TCH, _GC, _B)

    w1_2d = _packed_weights(W1)
    h1 = _tap_transform2(x, y, w1_2d)
    p1 = _sc_scatter(gE, wE, recv, h1, _C0, _C1)[:, :n, :]
    xl = _bn_relu(p1, bn1_scale, bn1_offset)

    w2_2d = _packed_weights(W2)
    h2 = _tap_transform(xl, w2_2d)
    p2 = _sc_scatter(gE, wE, recv, h2, _C0 + 2, _C1 - 2)[:, :n, :]
    return _finalize(p2, bn2_scale, bn2_offset, x, y)
